# baseline (speedup 1.0000x reference)
import functools

import jax
import jax.numpy as jnp
from jax import lax
from jax.experimental import pallas as pl
from jax.experimental.pallas import tpu as pltpu

TAU = 10.0
TOPK = 8
K = 64
D = 2048
TOKEN_BLOCK = 512


def _gate_block(z_ref, t_ref, cb_ref, e_ref, o_ref):
    z = z_ref[...]
    cb = cb_ref[...]

    raw = lax.dot_general(
        z, cb, (((1,), (1,)), ((), ())), preferred_element_type=jnp.float32)
    z2 = jnp.sum(z * z, axis=-1, keepdims=True)
    rz = TAU / jnp.maximum(jnp.sqrt(z2), 1e-12)
    c2 = lax.dot_general(
        jnp.ones((1, z.shape[1]), jnp.float32), cb * cb,
        (((1,), (1,)), ((), ())), preferred_element_type=jnp.float32)
    rc = 1.0 / jnp.maximum(jnp.sqrt(c2), 1e-12)
    logits = raw * rz * rc

    work = logits
    t = jnp.max(work, axis=-1, keepdims=True)
    mx = t
    for _ in range(TOPK - 1):
        work = jnp.where(work >= t, -jnp.inf, work)
        t = jnp.max(work, axis=-1, keepdims=True)

    e = jnp.where(logits >= t, jnp.exp(logits - mx), 0.0)
    w = e * (1.0 / jnp.sum(e, axis=-1, keepdims=True))

    g = lax.dot_general(
        w, e_ref[...], (((1,), (0,)), ((), ())), preferred_element_type=jnp.float32)
    o_ref[...] = t_ref[...] * (1.0 + g)


@functools.partial(jax.jit, static_argnames=())
def kernel(z, target, codebook, E):
    b, n, d = z.shape
    bn = b * n
    zf = z.reshape(bn, d)
    tf = target.reshape(bn, d)
    grid = (bn // TOKEN_BLOCK,)
    out = pl.pallas_call(
        _gate_block,
        grid=grid,
        in_specs=[
            pl.BlockSpec((TOKEN_BLOCK, d), lambda i: (i, 0)),
            pl.BlockSpec((TOKEN_BLOCK, d), lambda i: (i, 0)),
            pl.BlockSpec((K, d), lambda i: (0, 0)),
            pl.BlockSpec((K, d), lambda i: (0, 0)),
        ],
        out_specs=pl.BlockSpec((TOKEN_BLOCK, d), lambda i: (i, 0)),
        out_shape=jax.ShapeDtypeStruct((bn, d), jnp.float32),
        compiler_params=pltpu.CompilerParams(
            dimension_semantics=("arbitrary",),
        ),
    )(zf, tf, codebook, E)
    return out.reshape(b, n, d)


# speedup vs baseline: 1.0225x; 1.0225x over previous
import functools

import jax
import jax.numpy as jnp
from jax import lax
from jax.experimental import pallas as pl
from jax.experimental.pallas import tpu as pltpu

TAU = 10.0
TOPK = 8
K = 64
D = 2048
TOKEN_BLOCK = 1024


def _gate_block(z_ref, t_ref, cb_ref, e_ref, o_ref):
    z = z_ref[...]
    cb = cb_ref[...]

    raw = lax.dot_general(
        z, cb, (((1,), (1,)), ((), ())), preferred_element_type=jnp.float32)
    ones_col = jnp.ones((z.shape[1], 8), jnp.float32)
    z2 = lax.dot_general(
        z * z, ones_col, (((1,), (0,)), ((), ())),
        preferred_element_type=jnp.float32)[:, :1]
    rz = TAU / jnp.maximum(jnp.sqrt(z2), 1e-12)
    c2 = lax.dot_general(
        jnp.ones((1, z.shape[1]), jnp.float32), cb * cb,
        (((1,), (1,)), ((), ())), preferred_element_type=jnp.float32)
    rc = 1.0 / jnp.maximum(jnp.sqrt(c2), 1e-12)
    logits = raw * rz * rc

    work = logits
    t = jnp.max(work, axis=-1, keepdims=True)
    mx = t
    for _ in range(TOPK - 1):
        work = jnp.where(work >= t, -jnp.inf, work)
        t = jnp.max(work, axis=-1, keepdims=True)

    e = jnp.where(logits >= t, jnp.exp(logits - mx), 0.0)
    w = e * (1.0 / jnp.sum(e, axis=-1, keepdims=True))

    g = lax.dot_general(
        w, e_ref[...], (((1,), (0,)), ((), ())), preferred_element_type=jnp.float32)
    o_ref[...] = t_ref[...] * (1.0 + g)


@functools.partial(jax.jit, static_argnames=())
def kernel(z, target, codebook, E):
    b, n, d = z.shape
    bn = b * n
    zf = z.reshape(bn, d)
    tf = target.reshape(bn, d)
    grid = (bn // TOKEN_BLOCK,)
    out = pl.pallas_call(
        _gate_block,
        grid=grid,
        in_specs=[
            pl.BlockSpec((TOKEN_BLOCK, d), lambda i: (i, 0)),
            pl.BlockSpec((TOKEN_BLOCK, d), lambda i: (i, 0)),
            pl.BlockSpec((K, d), lambda i: (0, 0)),
            pl.BlockSpec((K, d), lambda i: (0, 0)),
        ],
        out_specs=pl.BlockSpec((TOKEN_BLOCK, d), lambda i: (i, 0)),
        out_shape=jax.ShapeDtypeStruct((bn, d), jnp.float32),
        compiler_params=pltpu.CompilerParams(
            dimension_semantics=("arbitrary",),
        ),
    )(zf, tf, codebook, E)
    return out.reshape(b, n, d)
